# baseline (speedup 1.0000x reference)
import functools

import jax
import jax.numpy as jnp
from jax import lax
from jax.experimental import pallas as pl
from jax.experimental.pallas import tpu as pltpu
from jax.experimental.pallas import tpu_sc as plsc

EMB_DIM = 64
LANE = 128
CR = 1
CHUNK = CR * LANE
N_BUF = 2


def kernel(data, table):
    B = data.shape[0] * data.shape[1]
    idx2d = data.reshape(B // LANE, LANE)
    tblv = table.reshape(table.shape[0] // 2, 2 * EMB_DIM)

    info = plsc.get_sparse_core_info()
    nw = info.num_cores * info.num_subcores
    nr_per_w = (B // LANE) // nw
    n_chunks = nr_per_w // CR

    mesh = plsc.VectorSubcoreMesh(core_axis_name="c", subcore_axis_name="s")

    @functools.partial(
        pl.kernel,
        mesh=mesh,
        out_type=jax.ShapeDtypeStruct((B, EMB_DIM), jnp.float32),
        scratch_types=[
            pltpu.VMEM((nr_per_w, LANE), jnp.int32),
            pltpu.VMEM((N_BUF, CR, LANE), jnp.int32),
            pltpu.VMEM((N_BUF * CHUNK,), jnp.int32),
            pltpu.VMEM((N_BUF, CHUNK, 2 * EMB_DIM), jnp.float32),
            pltpu.VMEM((N_BUF, CHUNK, EMB_DIM), jnp.float32),
            pltpu.SemaphoreType.DMA((N_BUF,)),
            pltpu.SemaphoreType.DMA((N_BUF,)),
        ],
        compiler_params=pltpu.CompilerParams(needs_layout_passes=False),
    )
    def run(idx_hbm, tbl_hbm, out_hbm, idx_all, qp, pb, rows, rows64, gsem, ssem):
        wid = lax.axis_index("s") * info.num_cores + lax.axis_index("c")
        row0 = wid * nr_per_w
        pltpu.sync_copy(idx_hbm.at[pl.ds(row0, nr_per_w)], idx_all)
        lanes = lax.iota(jnp.int32, 16)

        def prep(g, b):
            for j in range(CR):
                for m in range(LANE // 16):
                    s = idx_all[g * CR + j, pl.ds(m * 16, 16)]
                    qp[b, j, pl.ds(m * 16, 16)] = lax.shift_right_logical(s, 1)
                    pb[pl.ds(b * CHUNK + j * LANE + m * 16, 16)] = (
                        lax.shift_left(jnp.bitwise_and(s, 1), 6))

        def fire_gathers(b):
            for j in range(CR):
                pltpu.async_copy(
                    tbl_hbm.at[qp.at[b].at[j]],
                    rows.at[b].at[pl.ds(j * LANE, LANE)],
                    gsem.at[b],
                )

        def wait_gathers(b):
            for _ in range(CR):
                pltpu.make_async_copy(
                    tbl_hbm.at[qp.at[b].at[0]],
                    rows.at[b].at[pl.ds(0, LANE)],
                    gsem.at[b],
                ).wait()

        def select(b):
            bv = jnp.full((16,), b, jnp.int32)
            for m in range(CHUNK // 16):
                rowv = lanes + (m * 16)
                colb = plsc.load_gather(pb, [rowv + (b * CHUNK)])
                for c2 in range(EMB_DIM):
                    v = plsc.load_gather(rows, [bv, rowv, colb + c2])
                    plsc.store_scatter(
                        rows64,
                        [bv, rowv, jnp.full((16,), c2, jnp.int32)],
                        v,
                    )

        def start_store(g, b):
            r = (row0 + g * CR) * LANE
            pltpu.async_copy(
                rows64.at[b],
                out_hbm.at[pl.ds(r, CHUNK)],
                ssem.at[b],
            )

        def wait_store(b):
            pltpu.make_async_copy(
                rows64.at[b],
                out_hbm.at[pl.ds(0, CHUNK)],
                ssem.at[b],
            ).wait()

        for b in range(N_BUF):
            prep(b, b)
            fire_gathers(b)

        def body(i, carry):
            for b in range(N_BUF):
                g = i * N_BUF + b
                wait_gathers(b)
                select(b)
                start_store(g, b)
                nxt = g + N_BUF

                @pl.when(nxt < n_chunks)
                def _():
                    wait_store(b)
                    prep(nxt, b)
                    fire_gathers(b)

            return carry

        lax.fori_loop(0, n_chunks // N_BUF, body, 0)
        for b in range(N_BUF):
            wait_store(b)

    out = run(idx2d, tblv)
    return out.reshape(data.shape[0], data.shape[1], EMB_DIM)
